# baseline (speedup 1.0000x reference)
import functools

import jax
import jax.numpy as jnp
from jax import lax
from jax.experimental import pallas as pl
from jax.experimental.pallas import tpu as pltpu
from jax.experimental.pallas import tpu_sc as plsc


_ROWS = 65536
_COLS = 256
_N = _ROWS * _COLS
_NW = 32
_PER_W = _N // _NW
_CHUNK = 32768
_NCHUNK = _PER_W // _CHUNK
_NVEC = _CHUNK // 16

_mesh = plsc.VectorSubcoreMesh(core_axis_name="c", subcore_axis_name="s")


@functools.partial(
    pl.kernel,
    out_type=jax.ShapeDtypeStruct((_N,), jnp.float32),
    mesh=_mesh,
    scratch_types=[
        pltpu.VMEM((_CHUNK,), jnp.float32),
    ],
)
def _sc_add_one(x_hbm, o_hbm, buf):
    wid = lax.axis_index("s") * 2 + lax.axis_index("c")
    base = wid * _PER_W

    def _add_chunk(_, __):
        def body(i, carry):
            v = buf[pl.ds(i * 16, 16)]
            buf[pl.ds(i * 16, 16)] = v + 1.0
            return carry

        lax.fori_loop(0, _NVEC, body, 0)

    for g in range(_NCHUNK):
        off = base + g * _CHUNK
        pltpu.sync_copy(x_hbm.at[pl.ds(off, _CHUNK)], buf)
        _add_chunk(None, None)
        if g == 0:
            @pl.when(wid == 0)
            def _set_row1():
                for j in range(_COLS // 16):
                    buf[pl.ds(256 + j * 16, 16)] = jnp.full(
                        (16,), 2.0, jnp.float32
                    )

        pltpu.sync_copy(buf, o_hbm.at[pl.ds(off, _CHUNK)])


def kernel(x):
    m, n = x.shape
    out_flat = _sc_add_one(x.reshape(-1))
    return out_flat.reshape(m, n)


# speedup vs baseline: 1.9326x; 1.9326x over previous
import functools

import jax
import jax.numpy as jnp
from jax import lax
from jax.experimental import pallas as pl
from jax.experimental.pallas import tpu as pltpu
from jax.experimental.pallas import tpu_sc as plsc


_ROWS = 65536
_COLS = 256
_N = _ROWS * _COLS
_NW = 32
_PER_W = _N // _NW
_CHUNK = 16384
_NCHUNK = _PER_W // _CHUNK
_NBUF = 2
_UNROLL = 8
_NVEC = _CHUNK // 16

_mesh = plsc.VectorSubcoreMesh(core_axis_name="c", subcore_axis_name="s")


@functools.partial(
    pl.kernel,
    out_type=jax.ShapeDtypeStruct((_N,), jnp.float32),
    mesh=_mesh,
    scratch_types=[
        pltpu.VMEM((_CHUNK,), jnp.float32),
        pltpu.VMEM((_CHUNK,), jnp.float32),
        pltpu.VMEM((_CHUNK,), jnp.float32),
        pltpu.VMEM((_CHUNK,), jnp.float32),
        pltpu.SemaphoreType.DMA,
        pltpu.SemaphoreType.DMA,
        pltpu.SemaphoreType.DMA,
        pltpu.SemaphoreType.DMA,
    ],
)
def _sc_add_one(x_hbm, o_hbm, ib0, ib1, ob0, ob1, si0, si1, so0, so1):
    ibufs = [ib0, ib1]
    obufs = [ob0, ob1]
    sin = [si0, si1]
    sout = [so0, so1]
    wid = lax.axis_index("s") * 2 + lax.axis_index("c")
    base = wid * _PER_W

    for b in range(_NBUF):
        pltpu.async_copy(
            x_hbm.at[pl.ds(base + b * _CHUNK, _CHUNK)], ibufs[b], sin[b]
        )

    def outer(t, carry):
        g0 = t * _NBUF
        for b in range(_NBUF):
            off = base + (g0 + b) * _CHUNK
            pltpu.make_async_copy(
                x_hbm.at[pl.ds(off, _CHUNK)], ibufs[b], sin[b]
            ).wait()

            @pl.when(g0 > 0)
            def _wait_drain():
                pltpu.make_async_copy(
                    obufs[b], o_hbm.at[pl.ds(off, _CHUNK)], sout[b]
                ).wait()

            def cbody(i, c):
                s = i * (16 * _UNROLL)
                for j in range(_UNROLL):
                    o = s + j * 16
                    obufs[b][pl.ds(o, 16)] = ibufs[b][pl.ds(o, 16)] + 1.0
                return c

            lax.fori_loop(0, _NVEC // _UNROLL, cbody, 0)

            if b == 0:
                @pl.when(jnp.logical_and(wid == 0, g0 == 0))
                def _set_row1():
                    for j in range(_COLS // 16):
                        obufs[0][pl.ds(256 + j * 16, 16)] = jnp.full(
                            (16,), 2.0, jnp.float32
                        )

            pltpu.async_copy(obufs[b], o_hbm.at[pl.ds(off, _CHUNK)], sout[b])

            @pl.when(g0 + b + _NBUF < _NCHUNK)
            def _next_in():
                noff = base + (g0 + b + _NBUF) * _CHUNK
                pltpu.async_copy(
                    x_hbm.at[pl.ds(noff, _CHUNK)], ibufs[b], sin[b]
                )

        return carry

    lax.fori_loop(0, _NCHUNK // _NBUF, outer, 0)

    for b in range(_NBUF):
        off = base + (_NCHUNK - _NBUF + b) * _CHUNK
        pltpu.make_async_copy(
            obufs[b], o_hbm.at[pl.ds(off, _CHUNK)], sout[b]
        ).wait()


def kernel(x):
    m, n = x.shape
    out_flat = _sc_add_one(x.reshape(-1))
    return out_flat.reshape(m, n)
